# Initial kernel scaffold; baseline (speedup 1.0000x reference)
import jax
import jax.numpy as jnp
from jax.experimental import pallas as pl


def kernel(x, emb):
    raise NotImplementedError("write your pallas kernel here")


# speedup vs baseline: 2.7475x; 2.7475x over previous
import jax
import jax.numpy as jnp
from jax.experimental import pallas as pl


def _scale_body(emb_ref, out_ref, *, scale):
    out_ref[...] = emb_ref[...] * scale


def kernel(x, emb):
    seq_len = x.shape[1]
    dim = emb.shape[1]
    scale = dim ** (-0.5)
    blk = 512
    grid = (seq_len // blk,)
    import functools
    return pl.pallas_call(
        functools.partial(_scale_body, scale=scale),
        grid=grid,
        in_specs=[pl.BlockSpec((blk, dim), lambda i: (i, 0))],
        out_specs=pl.BlockSpec((blk, dim), lambda i: (i, 0)),
        out_shape=jax.ShapeDtypeStruct((seq_len, dim), emb.dtype),
    )(emb)


# speedup vs baseline: 3.0151x; 1.0974x over previous
import jax
import jax.numpy as jnp
from jax.experimental import pallas as pl


def _scale_body(emb_ref, out_ref, *, scale):
    out_ref[...] = emb_ref[...] * scale


def kernel(x, emb):
    seq_len = x.shape[1]
    dim = emb.shape[1]
    scale = dim ** (-0.5)
    blk = 1024
    grid = (seq_len // blk,)
    import functools
    return pl.pallas_call(
        functools.partial(_scale_body, scale=scale),
        grid=grid,
        in_specs=[pl.BlockSpec((blk, dim), lambda i: (i, 0))],
        out_specs=pl.BlockSpec((blk, dim), lambda i: (i, 0)),
        out_shape=jax.ShapeDtypeStruct((seq_len, dim), emb.dtype),
    )(emb)


# speedup vs baseline: 3.2436x; 1.0758x over previous
import jax
import jax.numpy as jnp
from jax.experimental import pallas as pl


def _scale_body(emb_ref, out_ref, *, scale):
    out_ref[...] = emb_ref[...] * scale


def kernel(x, emb):
    seq_len = x.shape[1]
    dim = emb.shape[1]
    scale = dim ** (-0.5)
    blk = 2048
    grid = (seq_len // blk,)
    import functools
    return pl.pallas_call(
        functools.partial(_scale_body, scale=scale),
        grid=grid,
        in_specs=[pl.BlockSpec((blk, dim), lambda i: (i, 0))],
        out_specs=pl.BlockSpec((blk, dim), lambda i: (i, 0)),
        out_shape=jax.ShapeDtypeStruct((seq_len, dim), emb.dtype),
    )(emb)
